# baseline (speedup 1.0000x reference)
import functools

import jax
import jax.numpy as jnp
from jax import lax
from jax.experimental import pallas as pl
from jax.experimental.pallas import tpu as pltpu
from jax.experimental.pallas import tpu_sc as plsc

N = 10000
D = 128
E = 320000
NPAD = 10240
NC, NS = 2, 16
NW = NC * NS
CH = 128
NITER = 80
NPRE = 2
EPT = NITER * CH
EPAD = EPT * NW
RPT = NPAD // NS
ZB = 64
BN_EPS = 1e-5


@functools.lru_cache(maxsize=None)
def _mesh():
    return plsc.VectorSubcoreMesh(core_axis_name="c", subcore_axis_name="s")


def _deg_body(dst_hbm, out_hbm, zb, ones_b, idxb, acc, sem, zsem):
    c = lax.axis_index("c")
    s = lax.axis_index("s")
    zvec = jnp.zeros((16,), jnp.float32)
    evec = jnp.where(lax.iota(jnp.int32, 16) == 0, 1.0, 0.0)

    def initrow(i, carry):
        zb[i, :] = zvec
        ones_b[i, :] = evec
        return carry

    lax.fori_loop(0, CH, initrow, 0)
    wid = c * NS + s
    cp_i = pltpu.async_copy(dst_hbm.at[wid], idxb, sem)
    for j in range(RPT // CH):
        pltpu.async_copy(zb, acc.at[pl.ds(s * RPT + j * CH, CH)], zsem)
    for j in range(RPT // CH):
        pltpu.make_async_copy(zb, acc.at[pl.ds(s * RPT + j * CH, CH)], zsem).wait()
    cp_i.wait()
    plsc.subcore_barrier()

    FD = 8

    def group(g, carry):
        for b in range(FD):
            pltpu.async_copy(ones_b, acc.at[idxb.at[g * FD + b]], sem, add=True)
        for b in range(FD):
            pltpu.make_async_copy(ones_b, acc.at[idxb.at[g * FD + b]], sem).wait()
        return carry

    lax.fori_loop(0, NITER // FD, group, 0)
    plsc.subcore_barrier()
    for j in range(RPT // CH):
        r0 = s * RPT + j * CH
        pltpu.sync_copy(acc.at[pl.ds(r0, CH)], out_hbm.at[c, pl.ds(r0, CH)])


@functools.lru_cache(maxsize=None)
def _deg_call():
    return pl.kernel(
        _deg_body,
        out_type=jax.ShapeDtypeStruct((NC, NPAD, 16), jnp.float32),
        mesh=_mesh(),
        scratch_types=[
            pltpu.VMEM((CH, 16), jnp.float32),
            pltpu.VMEM((CH, 16), jnp.float32),
            pltpu.VMEM((NITER + NPRE, CH), jnp.int32),
            pltpu.VMEM_SHARED((NPAD, 16), jnp.float32),
            pltpu.SemaphoreType.DMA,
            pltpu.SemaphoreType.DMA,
        ],
    )


def _scat_body(src_hbm, dst_hbm, p_hbm, out_hbm, zb,
               sidx0, didx0, sidx1, didx1, rows0, rows1, acc,
               isem, gsem0, gsem1, ssem, zsem):
    cc = lax.axis_index("c")
    ss = lax.axis_index("s")
    wid = cc * NS + ss
    zvec = jnp.zeros((16,), jnp.float32)

    def zrow(i, carry):
        for k in range(D // 16):
            zb[i, pl.ds(k * 16, 16)] = zvec
        return carry

    lax.fori_loop(0, ZB, zrow, 0)
    for j in range(RPT // ZB):
        pltpu.async_copy(zb, acc.at[pl.ds(ss * RPT + j * ZB, ZB)], zsem)
    for j in range(RPT // ZB):
        pltpu.make_async_copy(zb, acc.at[pl.ds(ss * RPT + j * ZB, ZB)], zsem).wait()
    plsc.subcore_barrier()

    sidx = (sidx0, sidx1)
    didx = (didx0, didx1)
    rows = (rows0, rows1)
    gsem = (gsem0, gsem1)

    def start_idx(c, b):
        pltpu.async_copy(src_hbm.at[wid, c], sidx[b], isem)
        pltpu.async_copy(dst_hbm.at[wid, c], didx[b], isem)

    def wait_idx(b):
        pltpu.make_async_copy(src_hbm.at[wid, 0], sidx[b], isem).wait()
        pltpu.make_async_copy(dst_hbm.at[wid, 0], didx[b], isem).wait()

    def start_gather(b):
        pltpu.async_copy(p_hbm.at[sidx[b]], rows[b], gsem[b])

    def wait_gather(b):
        pltpu.make_async_copy(p_hbm.at[sidx[b]], rows[b], gsem[b]).wait()

    def start_scat(b):
        pltpu.async_copy(rows[b], acc.at[didx[b]], ssem, add=True)

    def wait_scat(b):
        pltpu.make_async_copy(rows[b], acc.at[didx[b]], ssem).wait()

    start_idx(0, 0)
    start_idx(1, 1)
    wait_idx(0)
    start_gather(0)
    wait_idx(1)
    start_gather(1)
    wait_gather(0)
    start_scat(0)
    start_idx(2, 0)

    def group(g, carry):
        c = 2 * g + 1
        for b, nb in ((1, 0), (0, 1)):
            wait_idx(nb)
            wait_scat(nb)
            start_gather(nb)
            wait_gather(b)
            start_scat(b)
            start_idx(c + 2, b)
            c = c + 1
        return carry

    lax.fori_loop(0, (NITER - 2) // 2, group, 0)

    wait_idx(0)
    wait_scat(0)
    wait_gather(1)
    start_scat(1)
    wait_scat(1)
    plsc.subcore_barrier()
    for j in range(RPT // CH):
        r0 = ss * RPT + j * CH
        pltpu.sync_copy(acc.at[pl.ds(r0, CH)], out_hbm.at[cc, pl.ds(r0, CH)])


@functools.lru_cache(maxsize=None)
def _scat_call():
    return pl.kernel(
        _scat_body,
        out_type=jax.ShapeDtypeStruct((NC, NPAD, D), jnp.float32),
        mesh=_mesh(),
        scratch_types=[
            pltpu.VMEM((ZB, D), jnp.float32),
            pltpu.VMEM((CH,), jnp.int32),
            pltpu.VMEM((CH,), jnp.int32),
            pltpu.VMEM((CH,), jnp.int32),
            pltpu.VMEM((CH,), jnp.int32),
            pltpu.VMEM((CH, D), jnp.float32),
            pltpu.VMEM((CH, D), jnp.float32),
            pltpu.VMEM_SHARED((NPAD, D), jnp.float32),
            pltpu.SemaphoreType.DMA,
            pltpu.SemaphoreType.DMA,
            pltpu.SemaphoreType.DMA,
            pltpu.SemaphoreType.DMA,
            pltpu.SemaphoreType.DMA,
        ],
    )


def _mm_scale_body(degp_ref, x_ref, w_ref, p_ref, dinv_ref):
    dp = degp_ref[...]
    degsum = dp[0, :N, 0] + dp[1, :N, 0] + 1.0
    dinv = lax.rsqrt(degsum).reshape(N, 1)
    dinv_ref[...] = dinv
    p_ref[...] = (
        jnp.dot(x_ref[...], w_ref[...], preferred_element_type=jnp.float32) * dinv
    )


def _mid_body(s_ref, p_ref, dinv_ref, b_ref, g_ref, be_ref, w_ref, out_ref):
    sp = s_ref[...]
    dinv = dinv_ref[...]
    u = (sp[0, :N] + sp[1, :N] + p_ref[...]) * dinv + b_ref[...]
    mu = jnp.mean(u, axis=0)
    var = jnp.mean((u - mu) ** 2, axis=0)
    h = (u - mu) * lax.rsqrt(var + BN_EPS) * g_ref[...] + be_ref[...]
    h = jnp.maximum(h, 0.0)
    out_ref[...] = (
        jnp.dot(h, w_ref[...], preferred_element_type=jnp.float32) * dinv
    )


def _fin_body(s_ref, p_ref, dinv_ref, b_ref, g_ref, be_ref, out_ref):
    sp = s_ref[...]
    u = (sp[0, :N] + sp[1, :N] + p_ref[...]) * dinv_ref[...] + b_ref[...]
    mu = jnp.mean(u, axis=0)
    var = jnp.mean((u - mu) ** 2, axis=0)
    h = (u - mu) * lax.rsqrt(var + BN_EPS) * g_ref[...] + be_ref[...]
    out_ref[...] = jnp.maximum(h, 0.0)


def kernel(x, edge_index, W1, b1, g1, be1, W2, b2, g2, be2):
    src = edge_index[0].astype(jnp.int32)
    dst = edge_index[1].astype(jnp.int32)
    pad = EPAD - E
    src_p = jnp.concatenate([src, jnp.zeros((pad,), jnp.int32)])
    dst_p = jnp.concatenate([dst, jnp.full((pad,), N, jnp.int32)])
    src3 = jnp.concatenate(
        [src_p.reshape(NW, NITER, CH),
         jnp.zeros((NW, NPRE, CH), jnp.int32)], axis=1)
    dst3 = jnp.concatenate(
        [dst_p.reshape(NW, NITER, CH),
         jnp.full((NW, NPRE, CH), N, jnp.int32)], axis=1)

    degp = _deg_call()(dst3)

    p1, dinv = pl.pallas_call(
        _mm_scale_body,
        out_shape=(
            jax.ShapeDtypeStruct((N, D), jnp.float32),
            jax.ShapeDtypeStruct((N, 1), jnp.float32),
        ),
    )(degp, x, W1)

    s1 = _scat_call()(src3, dst3, p1)

    p2 = pl.pallas_call(
        _mid_body,
        out_shape=jax.ShapeDtypeStruct((N, D), jnp.float32),
    )(s1, p1, dinv, b1, g1, be1, W2)

    s2 = _scat_call()(src3, dst3, p2)

    out = pl.pallas_call(
        _fin_body,
        out_shape=jax.ShapeDtypeStruct((N, D), jnp.float32),
    )(s2, p2, dinv, b2, g2, be2)

    return out


# speedup vs baseline: 1.0249x; 1.0249x over previous
import functools

import jax
import jax.numpy as jnp
from jax import lax
from jax.experimental import pallas as pl
from jax.experimental.pallas import tpu as pltpu
from jax.experimental.pallas import tpu_sc as plsc

N = 10000
D = 128
E = 320000
NPAD = 10240
NC, NS = 2, 16
NW = NC * NS
CH = 128
NITER = 80
NPRE = 2
EPT = NITER * CH
EPAD = EPT * NW
RPT = NPAD // NS
ZB = 64
BN_EPS = 1e-5


@functools.lru_cache(maxsize=None)
def _mesh():
    return plsc.VectorSubcoreMesh(core_axis_name="c", subcore_axis_name="s")


def _deg_body(dst_hbm, out_hbm, zb, ones_b, idxb, acc, sem, zsem):
    c = lax.axis_index("c")
    s = lax.axis_index("s")
    zvec = jnp.zeros((16,), jnp.float32)
    evec = jnp.where(lax.iota(jnp.int32, 16) == 0, 1.0, 0.0)

    def initrow(i, carry):
        zb[i, :] = zvec
        ones_b[i, :] = evec
        return carry

    lax.fori_loop(0, CH, initrow, 0)
    wid = c * NS + s
    cp_i = pltpu.async_copy(dst_hbm.at[wid], idxb, sem)
    for j in range(RPT // CH):
        pltpu.async_copy(zb, acc.at[pl.ds(s * RPT + j * CH, CH)], zsem)
    for j in range(RPT // CH):
        pltpu.make_async_copy(zb, acc.at[pl.ds(s * RPT + j * CH, CH)], zsem).wait()
    cp_i.wait()
    plsc.subcore_barrier()

    FD = 8

    def group(g, carry):
        for b in range(FD):
            pltpu.async_copy(ones_b, acc.at[idxb.at[g * FD + b]], sem, add=True)
        for b in range(FD):
            pltpu.make_async_copy(ones_b, acc.at[idxb.at[g * FD + b]], sem).wait()
        return carry

    lax.fori_loop(0, NITER // FD, group, 0)
    plsc.subcore_barrier()
    for j in range(RPT // CH):
        r0 = s * RPT + j * CH
        pltpu.sync_copy(acc.at[pl.ds(r0, CH)], out_hbm.at[c, pl.ds(r0, CH)])


@functools.lru_cache(maxsize=None)
def _deg_call():
    return pl.kernel(
        _deg_body,
        out_type=jax.ShapeDtypeStruct((NC, NPAD, 16), jnp.float32),
        mesh=_mesh(),
        scratch_types=[
            pltpu.VMEM((CH, 16), jnp.float32),
            pltpu.VMEM((CH, 16), jnp.float32),
            pltpu.VMEM((NITER + NPRE, CH), jnp.int32),
            pltpu.VMEM_SHARED((NPAD, 16), jnp.float32),
            pltpu.SemaphoreType.DMA,
            pltpu.SemaphoreType.DMA,
        ],
    )


def _scat_body(src_hbm, dst_hbm, p_hbm, out_hbm, zb,
               sidx0, didx0, sidx1, didx1, rows0, rows1, acc,
               isem, gsem0, gsem1, ssem, zsem):
    cc = lax.axis_index("c")
    ss = lax.axis_index("s")
    wid = cc * NS + ss
    zvec = jnp.zeros((16,), jnp.float32)

    def zrow(i, carry):
        for k in range(D // 16):
            zb[i, pl.ds(k * 16, 16)] = zvec
        return carry

    lax.fori_loop(0, ZB, zrow, 0)
    for j in range(RPT // ZB):
        pltpu.async_copy(zb, acc.at[pl.ds(ss * RPT + j * ZB, ZB)], zsem)
    for j in range(RPT // ZB):
        pltpu.make_async_copy(zb, acc.at[pl.ds(ss * RPT + j * ZB, ZB)], zsem).wait()
    plsc.subcore_barrier()

    sidx = (sidx0, sidx1)
    didx = (didx0, didx1)
    rows = (rows0, rows1)
    gsem = (gsem0, gsem1)

    def start_idx(c, b):
        pltpu.async_copy(src_hbm.at[wid, c], sidx[b], isem)
        pltpu.async_copy(dst_hbm.at[wid, c], didx[b], isem)

    def wait_idx(b):
        pltpu.make_async_copy(src_hbm.at[wid, 0], sidx[b], isem).wait()
        pltpu.make_async_copy(dst_hbm.at[wid, 0], didx[b], isem).wait()

    def start_gather(b):
        pltpu.async_copy(p_hbm.at[sidx[b]], rows[b], gsem[b])

    def wait_gather(b):
        pltpu.make_async_copy(p_hbm.at[sidx[b]], rows[b], gsem[b]).wait()

    def start_scat(b):
        pltpu.async_copy(rows[b], acc.at[didx[b]], ssem, add=True)

    def wait_scat(b):
        pltpu.make_async_copy(rows[b], acc.at[didx[b]], ssem).wait()

    start_idx(0, 0)
    start_idx(1, 1)
    wait_idx(0)
    start_gather(0)
    wait_idx(1)
    start_gather(1)
    wait_gather(0)
    start_scat(0)
    start_idx(2, 0)

    def group(g, carry):
        c = 2 * g + 1
        for b, nb in ((1, 0), (0, 1)):
            wait_idx(nb)
            wait_scat(nb)
            start_gather(nb)
            wait_gather(b)
            start_scat(b)
            start_idx(c + 2, b)
            c = c + 1
        return carry

    lax.fori_loop(0, (NITER - 2) // 2, group, 0)

    wait_idx(0)
    wait_scat(0)
    wait_gather(1)
    start_scat(1)
    wait_scat(1)
    plsc.subcore_barrier()
    for j in range(RPT // CH):
        r0 = ss * RPT + j * CH
        pltpu.sync_copy(acc.at[pl.ds(r0, CH)], out_hbm.at[cc, pl.ds(r0, CH)])


@functools.lru_cache(maxsize=None)
def _scat_call():
    return pl.kernel(
        _scat_body,
        out_type=jax.ShapeDtypeStruct((NC, NPAD, D), jnp.float32),
        mesh=_mesh(),
        scratch_types=[
            pltpu.VMEM((ZB, D), jnp.float32),
            pltpu.VMEM((CH,), jnp.int32),
            pltpu.VMEM((CH,), jnp.int32),
            pltpu.VMEM((CH,), jnp.int32),
            pltpu.VMEM((CH,), jnp.int32),
            pltpu.VMEM((CH, D), jnp.float32),
            pltpu.VMEM((CH, D), jnp.float32),
            pltpu.VMEM_SHARED((NPAD, D), jnp.float32),
            pltpu.SemaphoreType.DMA,
            pltpu.SemaphoreType.DMA,
            pltpu.SemaphoreType.DMA,
            pltpu.SemaphoreType.DMA,
            pltpu.SemaphoreType.DMA,
        ],
    )


def _mm_scale_body(degp_ref, x_ref, w_ref, p_ref, dinv_ref):
    dp = degp_ref[...]
    degsum = dp[0, :N, 0] + dp[1, :N, 0] + 1.0
    dinv = lax.rsqrt(degsum).reshape(N, 1)
    dinv_ref[...] = dinv
    p_ref[...] = (
        jnp.dot(x_ref[...], w_ref[...], preferred_element_type=jnp.float32) * dinv
    )


def _mid_body(s_ref, p_ref, dinv_ref, b_ref, g_ref, be_ref, w_ref, out_ref):
    sp = s_ref[...]
    dinv = dinv_ref[...]
    u = (sp[0, :N] + sp[1, :N] + p_ref[...]) * dinv + b_ref[...]
    mu = jnp.mean(u, axis=0)
    var = jnp.mean((u - mu) ** 2, axis=0)
    h = (u - mu) * lax.rsqrt(var + BN_EPS) * g_ref[...] + be_ref[...]
    h = jnp.maximum(h, 0.0)
    out_ref[...] = (
        jnp.dot(h, w_ref[...], preferred_element_type=jnp.float32) * dinv
    )


def _fin_body(s_ref, p_ref, dinv_ref, b_ref, g_ref, be_ref, out_ref):
    sp = s_ref[...]
    u = (sp[0, :N] + sp[1, :N] + p_ref[...]) * dinv_ref[...] + b_ref[...]
    mu = jnp.mean(u, axis=0)
    var = jnp.mean((u - mu) ** 2, axis=0)
    h = (u - mu) * lax.rsqrt(var + BN_EPS) * g_ref[...] + be_ref[...]
    out_ref[...] = jnp.maximum(h, 0.0)


def kernel(x, edge_index, W1, b1, g1, be1, W2, b2, g2, be2):
    src = edge_index[0].astype(jnp.int32)
    dst = edge_index[1].astype(jnp.int32)
    pad = EPAD - E
    src_p = jnp.concatenate([src, jnp.zeros((pad,), jnp.int32)])
    junk = N + jnp.arange(pad, dtype=jnp.int32) % (NPAD - N)
    dst_p = jnp.concatenate([dst, junk])
    src3 = jnp.concatenate(
        [src_p.reshape(NW, NITER, CH),
         jnp.zeros((NW, NPRE, CH), jnp.int32)], axis=1)
    dst3 = jnp.concatenate(
        [dst_p.reshape(NW, NITER, CH),
         jnp.full((NW, NPRE, CH), N, jnp.int32)], axis=1)

    degp = _deg_call()(dst3)

    p1, dinv = pl.pallas_call(
        _mm_scale_body,
        out_shape=(
            jax.ShapeDtypeStruct((N, D), jnp.float32),
            jax.ShapeDtypeStruct((N, 1), jnp.float32),
        ),
    )(degp, x, W1)

    s1 = _scat_call()(src3, dst3, p1)

    p2 = pl.pallas_call(
        _mid_body,
        out_shape=jax.ShapeDtypeStruct((N, D), jnp.float32),
    )(s1, p1, dinv, b1, g1, be1, W2)

    s2 = _scat_call()(src3, dst3, p2)

    out = pl.pallas_call(
        _fin_body,
        out_shape=jax.ShapeDtypeStruct((N, D), jnp.float32),
    )(s2, p2, dinv, b2, g2, be2)

    return out


# speedup vs baseline: 3.6938x; 3.6042x over previous
import functools

import jax
import jax.numpy as jnp
from jax import lax
from jax.experimental import pallas as pl
from jax.experimental.pallas import tpu as pltpu
from jax.experimental.pallas import tpu_sc as plsc

N = 10000
D = 128
E = 320000
NPAD = 10240
NC, NS = 2, 16
NW = NC * NS
CH = 128
NITER = 80
NPRE = 2
EPT = NITER * CH
EPAD = EPT * NW
RPT = NPAD // NS
ZB = 64
BN_EPS = 1e-5


@functools.lru_cache(maxsize=None)
def _mesh():
    return plsc.VectorSubcoreMesh(core_axis_name="c", subcore_axis_name="s")


def _deg_body(dst_hbm, out_hbm, zb, ones_b, idxb, acc, sem, zsem):
    c = lax.axis_index("c")
    s = lax.axis_index("s")
    zvec = jnp.zeros((16,), jnp.float32)
    evec = jnp.where(lax.iota(jnp.int32, 16) == 0, 1.0, 0.0)

    def initrow(i, carry):
        zb[i, :] = zvec
        ones_b[i, :] = evec
        return carry

    lax.fori_loop(0, CH, initrow, 0)
    wid = c * NS + s
    cp_i = pltpu.async_copy(dst_hbm.at[wid], idxb, sem)
    for j in range(RPT // CH):
        pltpu.async_copy(zb, acc.at[pl.ds(s * RPT + j * CH, CH)], zsem)
    for j in range(RPT // CH):
        pltpu.make_async_copy(zb, acc.at[pl.ds(s * RPT + j * CH, CH)], zsem).wait()
    cp_i.wait()
    plsc.subcore_barrier()

    FD = 8

    def group(g, carry):
        for b in range(FD):
            pltpu.async_copy(ones_b, acc.at[idxb.at[g * FD + b]], sem, add=True)
        for b in range(FD):
            pltpu.make_async_copy(ones_b, acc.at[idxb.at[g * FD + b]], sem).wait()
        return carry

    lax.fori_loop(0, NITER // FD, group, 0)
    plsc.subcore_barrier()
    for j in range(RPT // CH):
        r0 = s * RPT + j * CH
        pltpu.sync_copy(acc.at[pl.ds(r0, CH)], out_hbm.at[c, pl.ds(r0, CH)])


@functools.lru_cache(maxsize=None)
def _deg_call():
    return pl.kernel(
        _deg_body,
        out_type=jax.ShapeDtypeStruct((NC, NPAD, 16), jnp.float32),
        mesh=_mesh(),
        scratch_types=[
            pltpu.VMEM((CH, 16), jnp.float32),
            pltpu.VMEM((CH, 16), jnp.float32),
            pltpu.VMEM((NITER + NPRE, CH), jnp.int32),
            pltpu.VMEM_SHARED((NPAD, 16), jnp.float32),
            pltpu.SemaphoreType.DMA,
            pltpu.SemaphoreType.DMA,
        ],
    )


def _scat_body(src_hbm, dst_hbm, p_hbm, out_hbm, zb,
               sidx0, didx0, sidx1, didx1, rows0, rows1, acc,
               isem, gsem0, gsem1, ssem, zsem):
    cc = lax.axis_index("c")
    ss = lax.axis_index("s")
    wid = cc * NS + ss
    zvec = jnp.zeros((16,), jnp.float32)

    def zrow(i, carry):
        for k in range(D // 16):
            zb[i, pl.ds(k * 16, 16)] = zvec
        return carry

    lax.fori_loop(0, ZB, zrow, 0)
    for j in range(RPT // ZB):
        pltpu.async_copy(zb, acc.at[pl.ds(ss * RPT + j * ZB, ZB)], zsem)
    for j in range(RPT // ZB):
        pltpu.make_async_copy(zb, acc.at[pl.ds(ss * RPT + j * ZB, ZB)], zsem).wait()
    plsc.subcore_barrier()

    sidx = (sidx0, sidx1)
    didx = (didx0, didx1)
    rows = (rows0, rows1)
    gsem = (gsem0, gsem1)

    def start_idx(c, b):
        pltpu.async_copy(src_hbm.at[wid, c], sidx[b], isem)
        pltpu.async_copy(dst_hbm.at[wid, c], didx[b], isem)

    def wait_idx(b):
        pltpu.make_async_copy(src_hbm.at[wid, 0], sidx[b], isem).wait()
        pltpu.make_async_copy(dst_hbm.at[wid, 0], didx[b], isem).wait()

    def start_gather(b):
        pltpu.async_copy(p_hbm.at[sidx[b]], rows[b], gsem[b])

    def wait_gather(b):
        pltpu.make_async_copy(p_hbm.at[sidx[b]], rows[b], gsem[b]).wait()

    def start_scat(b):
        pltpu.async_copy(rows[b], acc.at[didx[b]], ssem, add=True)

    def wait_scat(b):
        pltpu.make_async_copy(rows[b], acc.at[didx[b]], ssem).wait()

    start_idx(0, 0)
    start_idx(1, 1)
    wait_idx(0)
    start_gather(0)
    wait_idx(1)
    start_gather(1)
    wait_gather(0)
    start_scat(0)
    start_idx(2, 0)

    def group(g, carry):
        c = 2 * g + 1
        for b, nb in ((1, 0), (0, 1)):
            wait_idx(nb)
            wait_scat(nb)
            start_gather(nb)
            wait_gather(b)
            start_scat(b)
            start_idx(c + 2, b)
            c = c + 1
        return carry

    lax.fori_loop(0, (NITER - 2) // 2, group, 0)

    wait_idx(0)
    wait_scat(0)
    wait_gather(1)
    start_scat(1)
    wait_scat(1)
    plsc.subcore_barrier()
    for j in range(RPT // CH):
        r0 = ss * RPT + j * CH
        pltpu.sync_copy(acc.at[pl.ds(r0, CH)], out_hbm.at[cc, pl.ds(r0, CH)])


@functools.lru_cache(maxsize=None)
def _scat_call():
    return pl.kernel(
        _scat_body,
        out_type=jax.ShapeDtypeStruct((NC, NPAD, D), jnp.float32),
        mesh=_mesh(),
        scratch_types=[
            pltpu.VMEM((ZB, D), jnp.float32),
            pltpu.VMEM((CH,), jnp.int32),
            pltpu.VMEM((CH,), jnp.int32),
            pltpu.VMEM((CH,), jnp.int32),
            pltpu.VMEM((CH,), jnp.int32),
            pltpu.VMEM((CH, D), jnp.float32),
            pltpu.VMEM((CH, D), jnp.float32),
            pltpu.VMEM_SHARED((NPAD, D), jnp.float32),
            pltpu.SemaphoreType.DMA,
            pltpu.SemaphoreType.DMA,
            pltpu.SemaphoreType.DMA,
            pltpu.SemaphoreType.DMA,
            pltpu.SemaphoreType.DMA,
        ],
    )


def _mm_scale_body(degp_ref, x_ref, w_ref, p_ref, dinv_ref):
    dp = degp_ref[...]
    degsum = dp[0, :N, 0] + dp[1, :N, 0] + 1.0
    dinv = lax.rsqrt(degsum).reshape(N, 1)
    dinv_ref[...] = dinv
    p_ref[...] = (
        jnp.dot(x_ref[...], w_ref[...], preferred_element_type=jnp.float32) * dinv
    )


def _mid_body(s_ref, p_ref, dinv_ref, b_ref, g_ref, be_ref, w_ref, out_ref):
    sp = s_ref[...]
    dinv = dinv_ref[...]
    u = (sp[0, :N] + sp[1, :N] + p_ref[...]) * dinv + b_ref[...]
    mu = jnp.mean(u, axis=0)
    var = jnp.mean((u - mu) ** 2, axis=0)
    h = (u - mu) * lax.rsqrt(var + BN_EPS) * g_ref[...] + be_ref[...]
    h = jnp.maximum(h, 0.0)
    out_ref[...] = (
        jnp.dot(h, w_ref[...], preferred_element_type=jnp.float32) * dinv
    )


def _fin_body(s_ref, p_ref, dinv_ref, b_ref, g_ref, be_ref, out_ref):
    sp = s_ref[...]
    u = (sp[0, :N] + sp[1, :N] + p_ref[...]) * dinv_ref[...] + b_ref[...]
    mu = jnp.mean(u, axis=0)
    var = jnp.mean((u - mu) ** 2, axis=0)
    h = (u - mu) * lax.rsqrt(var + BN_EPS) * g_ref[...] + be_ref[...]
    out_ref[...] = jnp.maximum(h, 0.0)


def kernel(x, edge_index, W1, b1, g1, be1, W2, b2, g2, be2):
    src = edge_index[0].astype(jnp.int32)
    dst = edge_index[1].astype(jnp.int32)
    pad = EPAD - E
    jsrc = jnp.arange(pad, dtype=jnp.int32) % N
    src_p = jnp.concatenate([src, jsrc])
    junk = N + jnp.arange(pad, dtype=jnp.int32) % (NPAD - N)
    dst_p = jnp.concatenate([dst, junk])
    src3 = jnp.concatenate(
        [src_p.reshape(NW, NITER, CH),
         jnp.zeros((NW, NPRE, CH), jnp.int32)], axis=1)
    dst3 = jnp.concatenate(
        [dst_p.reshape(NW, NITER, CH),
         jnp.full((NW, NPRE, CH), N, jnp.int32)], axis=1)

    degp = _deg_call()(dst3)

    p1, dinv = pl.pallas_call(
        _mm_scale_body,
        out_shape=(
            jax.ShapeDtypeStruct((N, D), jnp.float32),
            jax.ShapeDtypeStruct((N, 1), jnp.float32),
        ),
    )(degp, x, W1)

    s1 = _scat_call()(src3, dst3, p1)

    p2 = pl.pallas_call(
        _mid_body,
        out_shape=jax.ShapeDtypeStruct((N, D), jnp.float32),
    )(s1, p1, dinv, b1, g1, be1, W2)

    s2 = _scat_call()(src3, dst3, p2)

    out = pl.pallas_call(
        _fin_body,
        out_shape=jax.ShapeDtypeStruct((N, D), jnp.float32),
    )(s2, p2, dinv, b2, g2, be2)

    return out
